# baseline (speedup 1.0000x reference)
import functools
import jax
import jax.numpy as jnp
from jax import lax
from jax.experimental import pallas as pl
from jax.experimental.pallas import tpu as pltpu
from jax.experimental.pallas import tpu_sc as plsc

N = 10000
E = 320000
D = 128
OUT_DIM = 128
SCALES = 3
PK = 128
NHALF = 2
EH = E // NHALF
EB = 3200
NW = 32
EPW = EH // NW
CH = 40
K = 5
GE = K * CH
NG = EPW // GE
NPAIR = NG // 2


def _sc_gather_body(table, src, dst, s_out, d_out,
                    idx_s, idx_d, rows, sem_g0, sem_g1, sem_w0, sem_w1):
    wid = lax.axis_index("s") * 2 + lax.axis_index("c")
    base0 = wid * EPW
    sem_g = (sem_g0, sem_g1)
    sem_w = (sem_w0, sem_w1)

    pltpu.sync_copy(src.at[pl.ds(base0, EPW)], idx_s)
    pltpu.sync_copy(dst.at[pl.ds(base0, EPW)], idx_d)

    def gathers(st, g):
        for k in range(K):
            off = g * GE + k * CH
            pltpu.async_copy(table.at[idx_s.at[pl.ds(off, CH)]],
                             rows.at[st, 2 * k], sem_g[st])
            pltpu.async_copy(table.at[idx_d.at[pl.ds(off, CH)]],
                             rows.at[st, 2 * k + 1], sem_g[st])

    def drain_g(st):
        for k in range(2 * K):
            pltpu.make_async_copy(
                table.at[idx_s.at[pl.ds(0, CH)]], rows.at[st, k],
                sem_g[st]).wait()

    def writes(st, g):
        for k in range(K):
            off = g * GE + k * CH
            pltpu.async_copy(rows.at[st, 2 * k],
                             s_out.at[pl.ds(base0 + off, CH)], sem_w[st])
            pltpu.async_copy(rows.at[st, 2 * k + 1],
                             d_out.at[pl.ds(base0 + off, CH)], sem_w[st])

    def drain_w(st, g):
        for k in range(K):
            off = g * GE + k * CH
            pltpu.make_async_copy(
                rows.at[st, 2 * k], s_out.at[pl.ds(base0 + off, CH)],
                sem_w[st]).wait()
            pltpu.make_async_copy(
                rows.at[st, 2 * k + 1], d_out.at[pl.ds(base0 + off, CH)],
                sem_w[st]).wait()

    gathers(0, 0)

    def body(j, _):
        g0 = 2 * j

        @pl.when(j > 0)
        def _():
            drain_w(1, g0 - 1)

        gathers(1, g0 + 1)
        drain_g(0)
        writes(0, g0)
        drain_w(0, g0)

        if NG % 2 == 0:
            @pl.when(j < NPAIR - 1)
            def _():
                gathers(0, g0 + 2)
        else:
            gathers(0, g0 + 2)

        drain_g(1)
        writes(1, g0 + 1)
        return 0

    lax.fori_loop(0, NPAIR, body, 0)
    if NG % 2 == 0:
        drain_w(1, NG - 1)
    else:
        drain_g(0)
        writes(0, NG - 1)
        drain_w(1, NG - 2)
        drain_w(0, NG - 1)


def _sc_gather(table, src, dst):
    mesh = plsc.VectorSubcoreMesh(core_axis_name="c", subcore_axis_name="s")
    fn = functools.partial(
        pl.kernel,
        mesh=mesh,
        out_type=[
            jax.ShapeDtypeStruct((EH, PK), jnp.int32),
            jax.ShapeDtypeStruct((EH, PK), jnp.int32),
        ],
        scratch_types=[
            pltpu.VMEM((EPW,), jnp.int32),
            pltpu.VMEM((EPW,), jnp.int32),
            pltpu.VMEM((2, 2 * K, CH, PK), jnp.int32),
            pltpu.SemaphoreType.DMA,
            pltpu.SemaphoreType.DMA,
            pltpu.SemaphoreType.DMA,
            pltpu.SemaphoreType.DMA,
        ],
    )(_sc_gather_body)
    return fn(table, src, dst)


def _mlp_block(s_ref, d_ref, w1_ref, b1_ref, w2_ref, b2_ref, o_ref):
    mm = lambda a, b: jax.lax.dot_general(
        a, b, (((1,), (0,)), ((), ())), preferred_element_type=jnp.float32)
    bf = jnp.bfloat16
    s = s_ref[...]
    d = d_ref[...]
    s_lo = lax.bitcast_convert_type(lax.shift_left(s, 16), jnp.float32)
    d_lo = lax.bitcast_convert_type(lax.shift_left(d, 16), jnp.float32)
    gap = (lax.bitcast_convert_type(s, jnp.float32)
           - lax.bitcast_convert_type(d, jnp.float32))
    x = jnp.concatenate(
        [s_lo.astype(bf), d_lo.astype(bf), gap.astype(bf)], axis=1)
    u = mm(x, w1_ref[...]) + b1_ref[...]
    h = jnp.maximum(u, 0.0).astype(bf)
    o_ref[...] = mm(h, w2_ref[...]) + b2_ref[...]


def _edge_mlp(s_pk, d_pk, w1c, b1c, w2s, b2m):
    grid = (EH // EB,)
    full = lambda shape: pl.BlockSpec(shape, lambda i: tuple(0 for _ in shape))
    return pl.pallas_call(
        _mlp_block,
        grid=grid,
        in_specs=[
            pl.BlockSpec((EB, PK), lambda i: (i, 0)),
            pl.BlockSpec((EB, PK), lambda i: (i, 0)),
            full((3 * D, SCALES * OUT_DIM)),
            full((1, SCALES * OUT_DIM)),
            full((SCALES * OUT_DIM, OUT_DIM)),
            full((1, OUT_DIM)),
        ],
        out_specs=pl.BlockSpec((EB, OUT_DIM), lambda i: (i, 0)),
        out_shape=jax.ShapeDtypeStruct((EH, OUT_DIM), jnp.float32),
    )(s_pk, d_pk, w1c, b1c, w2s, b2m)


def kernel(node_features, edge_index, temporal_info, W1, b1, W2, b2):
    f32, bf, i32 = jnp.float32, jnp.bfloat16, jnp.int32
    cat = lambda w: jnp.transpose(w, (1, 0, 2)).reshape(
        w.shape[1], SCALES * OUT_DIM)
    wa = cat(W1[:, :D, :])
    wb = cat(W1[:, D:2 * D, :])
    w_t = cat(W1[:, 2 * D:2 * D + 1, :])
    wt = jnp.concatenate([w_t, jnp.zeros((D - 1, SCALES * OUT_DIM), f32)],
                         axis=0)
    w1c = jnp.concatenate([wa, wb, wt], axis=0).astype(bf)
    b1c = b1.reshape(1, SCALES * OUT_DIM)
    w2s = (W2.reshape(SCALES * OUT_DIM, OUT_DIM) / SCALES).astype(bf)
    b2m = jnp.mean(b2, axis=0, keepdims=True)

    lo = lax.bitcast_convert_type(node_features.astype(bf), jnp.uint16)
    lo = lo.astype(i32)
    t_bits = lax.bitcast_convert_type(temporal_info.astype(bf), jnp.uint16)
    hi = jnp.concatenate(
        [t_bits.astype(i32)[:, None], jnp.zeros((N, D - 1), i32)], axis=1)
    table = jnp.bitwise_or(lo, lax.shift_left(hi, 16))

    eidx = edge_index.astype(i32)
    outs = []
    for h in range(NHALF):
        sl = slice(h * EH, (h + 1) * EH)
        s_pk, d_pk = _sc_gather(table, eidx[0, sl], eidx[1, sl])
        outs.append(_edge_mlp(s_pk, d_pk, w1c, b1c, w2s, b2m))
    return jnp.concatenate(outs, axis=0)


# speedup vs baseline: 1.1954x; 1.1954x over previous
import functools
import jax
import jax.numpy as jnp
from jax import lax
from jax.experimental import pallas as pl
from jax.experimental.pallas import tpu as pltpu
from jax.experimental.pallas import tpu_sc as plsc

N = 10000
E = 320000
D = 128
OUT_DIM = 128
SCALES = 3
PK = 128
NHALF = 2
EH = E // NHALF
EB = 3200
NW = 32
EPW = EH // NW
CH = 40
K = 5
GE = K * CH
NG = EPW // GE
NPAIR = NG // 2


def _sc_gather_body(table, src, dst, s_out, d_out,
                    idx_s, idx_d, rows, sem_g0, sem_g1, sem_w0, sem_w1):
    wid = lax.axis_index("s") * 2 + lax.axis_index("c")
    base0 = wid * EPW
    sem_g = (sem_g0, sem_g1)
    sem_w = (sem_w0, sem_w1)

    pltpu.sync_copy(src.at[pl.ds(base0, EPW)], idx_s)
    pltpu.sync_copy(dst.at[pl.ds(base0, EPW)], idx_d)

    def gathers(st, g):
        for k in range(K):
            off = g * GE + k * CH
            pltpu.async_copy(table.at[idx_s.at[pl.ds(off, CH)]],
                             rows.at[st, 2 * k], sem_g[st])
            pltpu.async_copy(table.at[idx_d.at[pl.ds(off, CH)]],
                             rows.at[st, 2 * k + 1], sem_g[st])

    def drain_g(st):
        for k in range(2 * K):
            pltpu.make_async_copy(
                table.at[idx_s.at[pl.ds(0, CH)]], rows.at[st, k],
                sem_g[st]).wait()

    def writes(st, g):
        for k in range(K):
            off = g * GE + k * CH
            pltpu.async_copy(rows.at[st, 2 * k],
                             s_out.at[pl.ds(base0 + off, CH)], sem_w[st])
            pltpu.async_copy(rows.at[st, 2 * k + 1],
                             d_out.at[pl.ds(base0 + off, CH)], sem_w[st])

    def drain_w(st, g):
        for k in range(K):
            off = g * GE + k * CH
            pltpu.make_async_copy(
                rows.at[st, 2 * k], s_out.at[pl.ds(base0 + off, CH)],
                sem_w[st]).wait()
            pltpu.make_async_copy(
                rows.at[st, 2 * k + 1], d_out.at[pl.ds(base0 + off, CH)],
                sem_w[st]).wait()

    gathers(0, 0)

    def body(j, _):
        g0 = 2 * j

        @pl.when(j > 0)
        def _():
            drain_w(1, g0 - 1)

        gathers(1, g0 + 1)
        drain_g(0)
        writes(0, g0)
        drain_w(0, g0)

        if NG % 2 == 0:
            @pl.when(j < NPAIR - 1)
            def _():
                gathers(0, g0 + 2)
        else:
            gathers(0, g0 + 2)

        drain_g(1)
        writes(1, g0 + 1)
        return 0

    lax.fori_loop(0, NPAIR, body, 0)
    if NG % 2 == 0:
        drain_w(1, NG - 1)
    else:
        drain_g(0)
        writes(0, NG - 1)
        drain_w(1, NG - 2)
        drain_w(0, NG - 1)


def _sc_gather(table, src, dst):
    mesh = plsc.VectorSubcoreMesh(core_axis_name="c", subcore_axis_name="s")
    fn = functools.partial(
        pl.kernel,
        mesh=mesh,
        out_type=[
            jax.ShapeDtypeStruct((EH, PK), jnp.int32),
            jax.ShapeDtypeStruct((EH, PK), jnp.int32),
        ],
        scratch_types=[
            pltpu.VMEM((EPW,), jnp.int32),
            pltpu.VMEM((EPW,), jnp.int32),
            pltpu.VMEM((2, 2 * K, CH, PK), jnp.int32),
            pltpu.SemaphoreType.DMA,
            pltpu.SemaphoreType.DMA,
            pltpu.SemaphoreType.DMA,
            pltpu.SemaphoreType.DMA,
        ],
    )(_sc_gather_body)
    return fn(table, src, dst)


def _mlp_block(s_ref, d_ref, w1_ref, b1_ref, w2_ref, b2_ref, o_ref):
    mm = lambda a, b: jax.lax.dot_general(
        a, b, (((1,), (0,)), ((), ())), preferred_element_type=jnp.float32)
    bf = jnp.bfloat16
    s = s_ref[...]
    d = d_ref[...]
    s_lo = lax.bitcast_convert_type(lax.shift_left(s, 16), jnp.float32)
    d_lo = lax.bitcast_convert_type(lax.shift_left(d, 16), jnp.float32)
    gap = (lax.bitcast_convert_type(s, jnp.float32)
           - lax.bitcast_convert_type(d, jnp.float32))
    x = jnp.concatenate(
        [s_lo.astype(bf), d_lo.astype(bf), gap.astype(bf)], axis=1)
    u = mm(x, w1_ref[...]) + b1_ref[...]
    h = jnp.maximum(u, 0.0).astype(bf)
    o_ref[...] = mm(h, w2_ref[...]) + b2_ref[...]


def _edge_mlp(h, prev, s_pk, d_pk, w1c, b1c, w2s, b2m):
    grid = (EH // EB,)
    base = h * (EH // EB)
    full = lambda shape: pl.BlockSpec(shape, lambda i: tuple(0 for _ in shape))
    in_specs = [
        pl.BlockSpec((EB, PK), lambda i: (i, 0)),
        pl.BlockSpec((EB, PK), lambda i: (i, 0)),
        full((3 * D, SCALES * OUT_DIM)),
        full((1, SCALES * OUT_DIM)),
        full((SCALES * OUT_DIM, OUT_DIM)),
        full((1, OUT_DIM)),
    ]
    args = (s_pk, d_pk, w1c, b1c, w2s, b2m)
    kwargs = {}
    body = _mlp_block
    if h > 0:
        def body(p_ref, *refs):
            _mlp_block(*refs)
        in_specs = [pl.BlockSpec(memory_space=pl.ANY)] + in_specs
        args = (prev,) + args
        kwargs["input_output_aliases"] = {0: 0}
    return pl.pallas_call(
        body,
        grid=grid,
        in_specs=in_specs,
        out_specs=pl.BlockSpec((EB, OUT_DIM), lambda i: (i + base, 0)),
        out_shape=jax.ShapeDtypeStruct((E, OUT_DIM), jnp.float32),
        **kwargs,
    )(*args)


def kernel(node_features, edge_index, temporal_info, W1, b1, W2, b2):
    f32, bf, i32 = jnp.float32, jnp.bfloat16, jnp.int32
    cat = lambda w: jnp.transpose(w, (1, 0, 2)).reshape(
        w.shape[1], SCALES * OUT_DIM)
    wa = cat(W1[:, :D, :])
    wb = cat(W1[:, D:2 * D, :])
    w_t = cat(W1[:, 2 * D:2 * D + 1, :])
    wt = jnp.concatenate([w_t, jnp.zeros((D - 1, SCALES * OUT_DIM), f32)],
                         axis=0)
    w1c = jnp.concatenate([wa, wb, wt], axis=0).astype(bf)
    b1c = b1.reshape(1, SCALES * OUT_DIM)
    w2s = (W2.reshape(SCALES * OUT_DIM, OUT_DIM) / SCALES).astype(bf)
    b2m = jnp.mean(b2, axis=0, keepdims=True)

    lo = lax.bitcast_convert_type(node_features.astype(bf), jnp.uint16)
    lo = lo.astype(i32)
    t_bits = lax.bitcast_convert_type(temporal_info.astype(bf), jnp.uint16)
    hi = jnp.concatenate(
        [t_bits.astype(i32)[:, None], jnp.zeros((N, D - 1), i32)], axis=1)
    table = jnp.bitwise_or(lo, lax.shift_left(hi, 16))

    eidx = edge_index.astype(i32)
    out = None
    for h in range(NHALF):
        sl = slice(h * EH, (h + 1) * EH)
        s_pk, d_pk = _sc_gather(table, eidx[0, sl], eidx[1, sl])
        out = _edge_mlp(h, out, s_pk, d_pk, w1c, b1c, w2s, b2m)
    return out
